# baseline (speedup 1.0000x reference)
import functools

import jax
import jax.numpy as jnp
from jax.experimental import pallas as pl


def _chamfer_body(p_ref, g_ref, d1_ref, d2_ref):
    i = pl.program_id(1)
    p = p_ref[0]
    g = g_ref[0]
    sq1 = jnp.sum(p * p, axis=0)
    sq2 = jnp.sum(g * g, axis=0)
    inner = jax.lax.dot_general(
        p, g, (((0,), (0,)), ((), ())), preferred_element_type=jnp.float32
    )
    d2 = sq1[:, None] + sq2[None, :] - 2.0 * inner
    d1_ref[0, 0, :] = jnp.min(d2, axis=1)
    colmin = jnp.min(d2, axis=0)

    @pl.when(i == 0)
    def _():
        d2_ref[0, 0, :] = colmin

    @pl.when(i > 0)
    def _():
        d2_ref[0, 0, :] = jnp.minimum(d2_ref[0, 0, :], colmin)


@functools.partial(jax.jit, static_argnames=("interpret",))
def kernel(pred, gt, interpret=False):
    B, N, _ = pred.shape
    M = gt.shape[1]
    predT = jnp.swapaxes(pred, 1, 2)
    gtT = jnp.swapaxes(gt, 1, 2)
    TN = 512
    dist1, dist2 = pl.pallas_call(
        _chamfer_body,
        grid=(B, N // TN),
        in_specs=[
            pl.BlockSpec((1, 3, TN), lambda b, i: (b, 0, i)),
            pl.BlockSpec((1, 3, M), lambda b, i: (b, 0, 0)),
        ],
        out_specs=[
            pl.BlockSpec((1, 1, TN), lambda b, i: (b, 0, i)),
            pl.BlockSpec((1, 1, M), lambda b, i: (b, 0, 0)),
        ],
        out_shape=[
            jax.ShapeDtypeStruct((B, 1, N), jnp.float32),
            jax.ShapeDtypeStruct((B, 1, M), jnp.float32),
        ],
        interpret=interpret,
    )(predT, gtT)
    return jnp.mean(dist1) + jnp.mean(dist2)
